# Initial kernel scaffold; baseline (speedup 1.0000x reference)
import jax
import jax.numpy as jnp
from jax.experimental import pallas as pl


def kernel(text, emb_table, W1, b1, W2, b2):
    raise NotImplementedError("write your pallas kernel here")


# speedup vs baseline: 9.2287x; 9.2287x over previous
import functools

import jax
import jax.numpy as jnp
from jax import lax
from jax.experimental import pallas as pl
from jax.experimental.pallas import tpu as pltpu
from jax.experimental.pallas import tpu_sc as plsc

B = 16384
L = 200
D = 32
HID = 512
NCLS = 1000

NC = 2
NS = 16
NW = NC * NS
BAGS_PER_W = B // NW
G = 4
CHUNK = 100
CPR = G * L // CHUNK
ROUNDS = BAGS_PER_W // G
TEXT_ROWS_PER_ROUND = G * L // CHUNK


def _sc_embedding_bag(text2d, emb_table):
    mesh = plsc.VectorSubcoreMesh(core_axis_name="c", subcore_axis_name="s")

    @functools.partial(
        pl.kernel,
        out_type=jax.ShapeDtypeStruct((B, D), jnp.float32),
        mesh=mesh,
        scratch_types=[
            pltpu.VMEM((CPR, CHUNK), jnp.int32),
            pltpu.VMEM((CPR, CHUNK, D), jnp.float32),
            pltpu.VMEM((G, D), jnp.float32),
            pltpu.SemaphoreType.DMA,
        ],
        compiler_params=pltpu.CompilerParams(use_tc_tiling_on_sc=False),
    )
    def body(text_hbm, table_hbm, out_hbm, idx_v, rows_v, emb_v, sem):
        cid = lax.axis_index("c")
        sid = lax.axis_index("s")
        wid = sid * NC + cid

        text_row0 = wid * (ROUNDS * TEXT_ROWS_PER_ROUND)
        out_row0 = wid * BAGS_PER_W

        def round_fn(r, carry):
            pltpu.sync_copy(
                text_hbm.at[pl.ds(text_row0 + r * TEXT_ROWS_PER_ROUND,
                                  TEXT_ROWS_PER_ROUND)],
                idx_v,
            )
            cps = [
                pltpu.async_copy(table_hbm.at[idx_v.at[j]], rows_v.at[j], sem)
                for j in range(CPR)
            ]
            for cp in cps:
                cp.wait()

            for b in range(G):
                def acc_chunk(c, acc):
                    def acc_row(rr, a):
                        a0, a1 = a
                        a0 = a0 + rows_v[c, rr, 0:16]
                        a1 = a1 + rows_v[c, rr, 16:32]
                        return (a0, a1)
                    return lax.fori_loop(0, CHUNK, acc_row, acc)

                acc = (jnp.zeros((16,), jnp.float32),
                       jnp.zeros((16,), jnp.float32))
                for c in range(2 * b, 2 * b + 2):
                    acc = acc_chunk(c, acc)
                emb_v[b, 0:16] = acc[0] * (1.0 / L)
                emb_v[b, 16:32] = acc[1] * (1.0 / L)

            pltpu.sync_copy(emb_v, out_hbm.at[pl.ds(out_row0 + r * G, G)])
            return carry

        lax.fori_loop(0, ROUNDS, round_fn, 0)

    return body(text2d, emb_table)


def _tc_mlp(x, W1, b1, W2, b2):
    BT = 1024
    grid = (B // BT,)

    def body(x_ref, w1_ref, b1_ref, w2_ref, b2_ref, o_ref):
        h = jnp.dot(x_ref[...], w1_ref[...],
                    preferred_element_type=jnp.float32) + b1_ref[...]
        h = jnp.maximum(h, 0.0)
        z = jnp.dot(h, w2_ref[...],
                    preferred_element_type=jnp.float32) + b2_ref[...]
        o_ref[...] = 1.0 / (1.0 + jnp.exp(-z))

    return pl.pallas_call(
        body,
        grid=grid,
        in_specs=[
            pl.BlockSpec((BT, D), lambda i: (i, 0)),
            pl.BlockSpec((D, HID), lambda i: (0, 0)),
            pl.BlockSpec((1, HID), lambda i: (0, 0)),
            pl.BlockSpec((HID, NCLS), lambda i: (0, 0)),
            pl.BlockSpec((1, NCLS), lambda i: (0, 0)),
        ],
        out_specs=pl.BlockSpec((BT, NCLS), lambda i: (i, 0)),
        out_shape=jax.ShapeDtypeStruct((B, NCLS), jnp.float32),
    )(x, W1, b1, W2, b2)


def kernel(text, emb_table, W1, b1, W2, b2):
    text2d = text.reshape(B * L // CHUNK, CHUNK).astype(jnp.int32)
    emb = _sc_embedding_bag(text2d, emb_table)
    return _tc_mlp(emb, W1, b1.reshape(1, HID), W2, b2.reshape(1, NCLS))


# speedup vs baseline: 14.5024x; 1.5715x over previous
import functools

import jax
import jax.numpy as jnp
from jax import lax
from jax.experimental import pallas as pl
from jax.experimental.pallas import tpu as pltpu
from jax.experimental.pallas import tpu_sc as plsc

B = 16384
L = 200
D = 32
HID = 512
NCLS = 1000

NC = 2
NS = 16
NW = NC * NS
BAGS_PER_W = B // NW
G = 4
CHUNK = 100
CPR = G * L // CHUNK
ROUNDS = BAGS_PER_W // G
TEXT_ROWS_PER_ROUND = G * L // CHUNK


def _sc_embedding_bag(text2d, emb_table):
    mesh = plsc.VectorSubcoreMesh(core_axis_name="c", subcore_axis_name="s")

    @functools.partial(
        pl.kernel,
        out_type=jax.ShapeDtypeStruct((B, D), jnp.float32),
        mesh=mesh,
        scratch_types=[
            pltpu.VMEM((2, CPR, CHUNK), jnp.int32),
            pltpu.VMEM((2, CPR, CHUNK, D), jnp.float32),
            pltpu.VMEM((BAGS_PER_W, D), jnp.float32),
            pltpu.SemaphoreType.DMA,
            pltpu.SemaphoreType.DMA,
            pltpu.SemaphoreType.DMA,
            pltpu.SemaphoreType.DMA,
        ],
        compiler_params=pltpu.CompilerParams(use_tc_tiling_on_sc=False),
    )
    def body(text_hbm, table_hbm, out_hbm, idx_v, rows_v, emb_v,
             rs0, rs1, is0, is1):
        cid = lax.axis_index("c")
        sid = lax.axis_index("s")
        wid = sid * NC + cid
        rsem = [rs0, rs1]
        isem = [is0, is1]

        text_row0 = wid * (ROUNDS * TEXT_ROWS_PER_ROUND)
        out_row0 = wid * BAGS_PER_W

        def idx_src(r):
            return text_hbm.at[pl.ds(text_row0 + r * TEXT_ROWS_PER_ROUND,
                                     TEXT_ROWS_PER_ROUND)]

        def fire_gathers(bank):
            for j in range(CPR):
                pltpu.async_copy(table_hbm.at[idx_v.at[bank, j]],
                                 rows_v.at[bank, j], rsem[bank])

        def drain_gathers(bank):
            for j in range(CPR):
                pltpu.make_async_copy(table_hbm.at[idx_v.at[bank, j]],
                                      rows_v.at[bank, j], rsem[bank]).wait()

        def accumulate(bank, r):
            zero = jnp.zeros((16,), jnp.float32)
            init = (zero,) * (2 * CPR)

            def acc_row(rr, accs):
                accs = list(accs)
                for c in range(CPR):
                    for h in range(2):
                        v = rows_v[bank, c, rr, 16 * h:16 * h + 16]
                        accs[2 * c + h] = accs[2 * c + h] + v
                return tuple(accs)

            accs = lax.fori_loop(0, CHUNK, acc_row, init, unroll=2)
            for b in range(G):
                a0 = (accs[4 * b + 0] + accs[4 * b + 2]) * (1.0 / L)
                a1 = (accs[4 * b + 1] + accs[4 * b + 3]) * (1.0 / L)
                emb_v[r * G + b, 0:16] = a0
                emb_v[r * G + b, 16:32] = a1

        def phase(r, a, b):
            @pl.when(r + 1 < ROUNDS)
            def _():
                pltpu.make_async_copy(idx_src(r + 1), idx_v.at[b],
                                      isem[b]).wait()
                fire_gathers(b)
            drain_gathers(a)

            @pl.when(r + 2 < ROUNDS)
            def _():
                pltpu.async_copy(idx_src(r + 2), idx_v.at[a], isem[a])

            accumulate(a, r)

        pltpu.async_copy(idx_src(0), idx_v.at[0], is0).wait()
        fire_gathers(0)
        pltpu.async_copy(idx_src(1), idx_v.at[1], is1)

        def gbody(g, carry):
            phase(2 * g, 0, 1)
            phase(2 * g + 1, 1, 0)
            return carry

        lax.fori_loop(0, ROUNDS // 2, gbody, 0)
        pltpu.sync_copy(emb_v, out_hbm.at[pl.ds(out_row0, BAGS_PER_W)])

    return body(text2d, emb_table)


def _tc_mlp(x, W1, b1, W2, b2):
    BT = 1024
    grid = (B // BT,)

    def body(x_ref, w1_ref, b1_ref, w2_ref, b2_ref, o_ref):
        h = jnp.dot(x_ref[...], w1_ref[...],
                    preferred_element_type=jnp.float32) + b1_ref[...]
        h = jnp.maximum(h, 0.0)
        z = jnp.dot(h, w2_ref[...],
                    preferred_element_type=jnp.float32) + b2_ref[...]
        o_ref[...] = 1.0 / (1.0 + jnp.exp(-z))

    return pl.pallas_call(
        body,
        grid=grid,
        in_specs=[
            pl.BlockSpec((BT, D), lambda i: (i, 0)),
            pl.BlockSpec((D, HID), lambda i: (0, 0)),
            pl.BlockSpec((1, HID), lambda i: (0, 0)),
            pl.BlockSpec((HID, NCLS), lambda i: (0, 0)),
            pl.BlockSpec((1, NCLS), lambda i: (0, 0)),
        ],
        out_specs=pl.BlockSpec((BT, NCLS), lambda i: (i, 0)),
        out_shape=jax.ShapeDtypeStruct((B, NCLS), jnp.float32),
    )(x, W1, b1, W2, b2)


def kernel(text, emb_table, W1, b1, W2, b2):
    text2d = text.reshape(B * L // CHUNK, CHUNK).astype(jnp.int32)
    emb = _sc_embedding_bag(text2d, emb_table)
    return _tc_mlp(emb, W1, b1.reshape(1, HID), W2, b2.reshape(1, NCLS))


# speedup vs baseline: 14.6946x; 1.0133x over previous
import functools

import jax
import jax.numpy as jnp
from jax import lax
from jax.experimental import pallas as pl
from jax.experimental.pallas import tpu as pltpu
from jax.experimental.pallas import tpu_sc as plsc

B = 16384
L = 200
D = 32
HID = 512
NCLS = 1000

NC = 2
NS = 16
NW = NC * NS
BAGS_PER_W = B // NW
G = 4
CHUNK = 100
CPR = G * L // CHUNK
ROUNDS = BAGS_PER_W // G
TEXT_ROWS_PER_ROUND = G * L // CHUNK


def _sc_embedding_bag(text, emb_table):
    mesh = plsc.VectorSubcoreMesh(core_axis_name="c", subcore_axis_name="s")

    SPLITS = ((0, 104), (104, 96))

    @functools.partial(
        pl.kernel,
        out_type=jax.ShapeDtypeStruct((B, D), jnp.float32),
        mesh=mesh,
        scratch_types=[
            pltpu.VMEM((2, G, L), jnp.int32),
            pltpu.VMEM((2, G, L, D), jnp.float32),
            pltpu.VMEM((BAGS_PER_W, D), jnp.float32),
            pltpu.SemaphoreType.DMA,
            pltpu.SemaphoreType.DMA,
            pltpu.SemaphoreType.DMA,
            pltpu.SemaphoreType.DMA,
        ],
        compiler_params=pltpu.CompilerParams(use_tc_tiling_on_sc=False),
    )
    def body(text_hbm, table_hbm, out_hbm, idx_v, rows_v, emb_v,
             rs0, rs1, is0, is1):
        cid = lax.axis_index("c")
        sid = lax.axis_index("s")
        wid = sid * NC + cid
        rsem = [rs0, rs1]
        isem = [is0, is1]

        out_row0 = wid * BAGS_PER_W

        def idx_src(r):
            return text_hbm.at[pl.ds(out_row0 + r * G, G)]

        def fire_gathers(bank):
            for b in range(G):
                for off, sz in SPLITS:
                    pltpu.async_copy(
                        table_hbm.at[idx_v.at[bank, b, pl.ds(off, sz)]],
                        rows_v.at[bank, b, pl.ds(off, sz)], rsem[bank])

        def drain_gathers(bank):
            for b in range(G):
                for off, sz in SPLITS:
                    pltpu.make_async_copy(
                        table_hbm.at[idx_v.at[bank, b, pl.ds(off, sz)]],
                        rows_v.at[bank, b, pl.ds(off, sz)],
                        rsem[bank]).wait()

        def accumulate(bank, r):
            zero = jnp.zeros((16,), jnp.float32)
            init = (zero,) * (2 * G)

            def acc_row(rr, accs):
                accs = list(accs)
                for b in range(G):
                    for h in range(2):
                        v = rows_v[bank, b, rr, 16 * h:16 * h + 16]
                        accs[2 * b + h] = accs[2 * b + h] + v
                return tuple(accs)

            accs = lax.fori_loop(0, L, acc_row, init, unroll=2)
            for b in range(G):
                emb_v[r * G + b, 0:16] = accs[2 * b] * (1.0 / L)
                emb_v[r * G + b, 16:32] = accs[2 * b + 1] * (1.0 / L)

        def phase(r, a, b):
            @pl.when(r + 1 < ROUNDS)
            def _():
                pltpu.make_async_copy(idx_src(r + 1), idx_v.at[b],
                                      isem[b]).wait()
                fire_gathers(b)
            drain_gathers(a)

            @pl.when(r + 2 < ROUNDS)
            def _():
                pltpu.async_copy(idx_src(r + 2), idx_v.at[a], isem[a])

            accumulate(a, r)

        pltpu.async_copy(idx_src(0), idx_v.at[0], is0).wait()
        fire_gathers(0)
        pltpu.async_copy(idx_src(1), idx_v.at[1], is1)

        def gbody(g, carry):
            phase(2 * g, 0, 1)
            phase(2 * g + 1, 1, 0)
            return carry

        lax.fori_loop(0, ROUNDS // 2, gbody, 0)
        pltpu.sync_copy(emb_v, out_hbm.at[pl.ds(out_row0, BAGS_PER_W)])

    return body(text, emb_table)


def _tc_mlp(x, W1, b1, W2, b2):
    BT = 1024
    grid = (B // BT,)

    def body(x_ref, w1_ref, b1_ref, w2_ref, b2_ref, o_ref):
        h = jnp.dot(x_ref[...], w1_ref[...],
                    preferred_element_type=jnp.float32) + b1_ref[...]
        h = jnp.maximum(h, 0.0)
        z = jnp.dot(h, w2_ref[...],
                    preferred_element_type=jnp.float32) + b2_ref[...]
        o_ref[...] = 1.0 / (1.0 + jnp.exp(-z))

    return pl.pallas_call(
        body,
        grid=grid,
        in_specs=[
            pl.BlockSpec((BT, D), lambda i: (i, 0)),
            pl.BlockSpec((D, HID), lambda i: (0, 0)),
            pl.BlockSpec((1, HID), lambda i: (0, 0)),
            pl.BlockSpec((HID, NCLS), lambda i: (0, 0)),
            pl.BlockSpec((1, NCLS), lambda i: (0, 0)),
        ],
        out_specs=pl.BlockSpec((BT, NCLS), lambda i: (i, 0)),
        out_shape=jax.ShapeDtypeStruct((B, NCLS), jnp.float32),
    )(x, W1, b1, W2, b2)


def kernel(text, emb_table, W1, b1, W2, b2):
    emb = _sc_embedding_bag(text, emb_table)
    return _tc_mlp(emb, W1, b1.reshape(1, HID), W2, b2.reshape(1, NCLS))
